# Initial kernel scaffold; baseline (speedup 1.0000x reference)
import jax
import jax.numpy as jnp
from jax.experimental import pallas as pl


def kernel(x, params):
    raise NotImplementedError("write your pallas kernel here")


# speedup vs baseline: 4.2496x; 4.2496x over previous
import functools

import numpy as np
import jax
import jax.numpy as jnp
from jax.experimental import pallas as pl
from jax.experimental.pallas import tpu as pltpu

_D = 768
_H = 12
_DH = 64
_DFF = 3072
_FACTOR = 5

_SAMPLE_CACHE = {}


def _sampling(L):
    if L not in _SAMPLE_CACHE:
        U = min(_FACTOR * int(np.ceil(np.log(L))), L)
        with jax.ensure_compile_time_eval():
            skey = jax.random.fold_in(jax.random.key(42), L)
            idx = jax.random.randint(skey, (L, U), 0, L)
            cnt = jnp.zeros((L, L), jnp.float32).at[
                jnp.arange(L)[:, None], idx].add(1.0)
            _SAMPLE_CACHE[L] = (np.asarray(cnt), U)
    return _SAMPLE_CACHE[L]


def _mm_bias_body(x_ref, w_ref, b_ref, o_ref, *, act):
    y = jnp.dot(x_ref[...], w_ref[...], preferred_element_type=jnp.float32)
    y = y + b_ref[...]
    if act == "gelu":
        y = jax.nn.gelu(y)
    o_ref[...] = y


def _mm_bias(x, w, b, act=None, bq=256):
    L, K = x.shape
    N = w.shape[1]
    return pl.pallas_call(
        functools.partial(_mm_bias_body, act=act),
        grid=(L // bq,),
        in_specs=[
            pl.BlockSpec((bq, K), lambda i: (i, 0)),
            pl.BlockSpec((K, N), lambda i: (0, 0)),
            pl.BlockSpec((1, N), lambda i: (0, 0)),
        ],
        out_specs=pl.BlockSpec((bq, N), lambda i: (i, 0)),
        out_shape=jax.ShapeDtypeStruct((L, N), jnp.float32),
    )(x, w, b.reshape(1, N))


def _mm_res_ln_body(h_ref, w_ref, b_ref, r_ref, g_ref, bb_ref, o_ref):
    y = jnp.dot(h_ref[...], w_ref[...], preferred_element_type=jnp.float32)
    y = y + b_ref[...] + r_ref[...]
    m = jnp.mean(y, axis=1, keepdims=True)
    v = jnp.mean((y - m) * (y - m), axis=1, keepdims=True)
    o_ref[...] = (y - m) * jax.lax.rsqrt(v + 1e-5) * g_ref[...] + bb_ref[...]


def _mm_res_ln(h, w, b, res, g, bb, bq=256):
    L, K = h.shape
    N = w.shape[1]
    return pl.pallas_call(
        _mm_res_ln_body,
        grid=(L // bq,),
        in_specs=[
            pl.BlockSpec((bq, K), lambda i: (i, 0)),
            pl.BlockSpec((K, N), lambda i: (0, 0)),
            pl.BlockSpec((1, N), lambda i: (0, 0)),
            pl.BlockSpec((bq, N), lambda i: (i, 0)),
            pl.BlockSpec((1, N), lambda i: (0, 0)),
            pl.BlockSpec((1, N), lambda i: (0, 0)),
        ],
        out_specs=pl.BlockSpec((bq, N), lambda i: (i, 0)),
        out_shape=jax.ShapeDtypeStruct((L, N), jnp.float32),
    )(h, w, b.reshape(1, N), res, g.reshape(1, N), bb.reshape(1, N))


def _m_body(q_ref, k_ref, c_ref, m_ref, *, L):
    c = c_ref[...]
    msk = c > 0.0
    inv_l = 1.0 / L
    for h in range(_H):
        s = jax.lax.dot_general(
            q_ref[h], k_ref[h], (((1,), (1,)), ((), ())),
            preferred_element_type=jnp.float32)
        smax = jnp.max(jnp.where(msk, s, -jnp.inf), axis=1)
        ssum = jnp.sum(s * c, axis=1)
        m_ref[h, :] = smax - ssum * inv_l


def _m_stat(q, k, cnt, bq=256):
    L = q.shape[1]
    return pl.pallas_call(
        functools.partial(_m_body, L=L),
        grid=(L // bq,),
        in_specs=[
            pl.BlockSpec((_H, bq, _DH), lambda i: (0, i, 0)),
            pl.BlockSpec((_H, L, _DH), lambda i: (0, 0, 0)),
            pl.BlockSpec((bq, L), lambda i: (i, 0)),
        ],
        out_specs=pl.BlockSpec((_H, bq), lambda i: (0, i)),
        out_shape=jax.ShapeDtypeStruct((_H, L), jnp.float32),
    )(q, k, cnt)


def _topk_body(m_ref, o_ref, *, u, L):
    m = m_ref[...]
    iota = jax.lax.broadcasted_iota(jnp.int32, (_H, L), 1)
    cols = jax.lax.broadcasted_iota(jnp.int32, (_H, u), 1)

    def body(i, carry):
        m, top = carry
        mx = jnp.max(m, axis=1, keepdims=True)
        idx = jnp.min(jnp.where(m == mx, iota, L), axis=1, keepdims=True)
        top = jnp.where(cols == i, idx, top)
        m = jnp.where(iota == idx, -jnp.inf, m)
        return m, top

    _, top = jax.lax.fori_loop(
        0, u, body, (m, jnp.zeros((_H, u), jnp.int32)))
    o_ref[...] = top


def _topk(m, u):
    L = m.shape[1]
    return pl.pallas_call(
        functools.partial(_topk_body, u=u, L=L),
        grid=(1,),
        in_specs=[pl.BlockSpec((_H, L), lambda i: (0, 0))],
        out_specs=pl.BlockSpec((_H, u), lambda i: (0, 0)),
        out_shape=jax.ShapeDtypeStruct((_H, u), jnp.int32),
    )(m)


def _attn_body(q_ref, k_ref, v_ref, tc_ref, tr_ref, o_ref, *, u, L):
    q = q_ref[0]
    k = k_ref[0]
    v = v_ref[0]
    idx_col = tc_ref[0]
    idx_row = tr_ref[0]
    iota_ul = jax.lax.broadcasted_iota(jnp.int32, (u, L), 1)
    sel = (iota_ul == idx_col).astype(jnp.float32)
    qr = jnp.dot(sel, q, preferred_element_type=jnp.float32)
    s = jax.lax.dot_general(
        qr, k, (((1,), (1,)), ((), ())),
        preferred_element_type=jnp.float32) * (1.0 / np.sqrt(_DH))
    s = s - jnp.max(s, axis=1, keepdims=True)
    p = jnp.exp(s)
    attn = p / jnp.sum(p, axis=1, keepdims=True)
    av = jnp.dot(attn, v, preferred_element_type=jnp.float32)
    vmean = jnp.mean(v, axis=0, keepdims=True)
    iota_lu = jax.lax.broadcasted_iota(jnp.int32, (L, u), 1)
    sel_t = (iota_lu == idx_row).astype(jnp.float32)
    hit = jnp.sum(sel_t, axis=1, keepdims=True)
    o_ref[0] = jnp.dot(sel_t, av, preferred_element_type=jnp.float32) \
        + (1.0 - hit) * vmean


def _attn_ctx(q, k, v, top, u):
    L = q.shape[1]
    top_c = top.reshape(_H, u, 1)
    top_r = top.reshape(_H, 1, u)
    return pl.pallas_call(
        functools.partial(_attn_body, u=u, L=L),
        grid=(_H,),
        in_specs=[
            pl.BlockSpec((1, L, _DH), lambda h: (h, 0, 0)),
            pl.BlockSpec((1, L, _DH), lambda h: (h, 0, 0)),
            pl.BlockSpec((1, L, _DH), lambda h: (h, 0, 0)),
            pl.BlockSpec((1, u, 1), lambda h: (h, 0, 0)),
            pl.BlockSpec((1, 1, u), lambda h: (h, 0, 0)),
        ],
        out_specs=pl.BlockSpec((1, L, _DH), lambda h: (h, 0, 0)),
        out_shape=jax.ShapeDtypeStruct((_H, L, _DH), jnp.float32),
    )(q, k, v, top_c, top_r)


def _layer(x, lp):
    L = x.shape[0]
    cnt_np, u = _sampling(L)
    cnt = jnp.asarray(cnt_np)

    wqkv = jnp.concatenate([lp["Wq"], lp["Wk"], lp["Wv"]], axis=1)
    bqkv = jnp.concatenate([lp["bq"], lp["bk"], lp["bv"]])
    qkv = _mm_bias(x, wqkv, bqkv)
    q = qkv[:, :_D].reshape(L, _H, _DH).transpose(1, 0, 2)
    k = qkv[:, _D:2 * _D].reshape(L, _H, _DH).transpose(1, 0, 2)
    v = qkv[:, 2 * _D:].reshape(L, _H, _DH).transpose(1, 0, 2)

    m = _m_stat(q, k, cnt)
    top = _topk(m, u)
    ctx = _attn_ctx(q, k, v, top, u)
    ctx_flat = ctx.transpose(1, 0, 2).reshape(L, _D)

    x1 = _mm_res_ln(ctx_flat, lp["Wo"], lp["bo"], x, lp["g1"], lp["b1"])
    hh = _mm_bias(x1, lp["W1"], lp["bc1"], act="gelu")
    x2 = _mm_res_ln(hh, lp["W2"], lp["bc2"], x1, lp["g2"], lp["b2"])
    return x2


def kernel(x, params):
    layers = params["layers"]
    n_layers = len(layers)
    h = x[0]
    for i, lp in enumerate(layers):
        h = _layer(h, lp)
        if i < n_layers - 1:
            h = h[::2, :]
    return h[None]
